# baseline (speedup 1.0000x reference)
import functools

import jax
import jax.numpy as jnp
from jax import lax
from jax.experimental import pallas as pl
from jax.experimental.pallas import tpu as pltpu
from jax.experimental.pallas import tpu_sc as plsc

N = 10000
E = 320000
F = 128
B = 128
NC = 2
NS = 16
NW = NC * NS
EPW = E // NW
K = 80
NCH = EPW // K
RPS = N // NS
NJUNK = N + 16
RBLK = 2000
G = N // RBLK

_mesh = plsc.VectorSubcoreMesh(core_axis_name="c", subcore_axis_name="s")
_sc_params = pltpu.CompilerParams(
    needs_layout_passes=False, use_tc_tiling_on_sc=False
)


@functools.partial(
    pl.kernel,
    out_type=jax.ShapeDtypeStruct((N // RBLK, NW, RBLK), jnp.float32),
    mesh=_mesh,
    compiler_params=_sc_params,
    scratch_types=[
        pltpu.VMEM((NCH, K), jnp.int32),
        pltpu.VMEM((NCH, K), jnp.int32),
        pltpu.VMEM((N,), jnp.float32),
    ],
)
def _deg_kernel(row_hbm, col_hbm, dall_hbm, ridx, cidx, deg):
    c = lax.axis_index("c")
    s = lax.axis_index("s")
    w = s * NC + c
    pltpu.sync_copy(row_hbm.at[w], ridx)
    pltpu.sync_copy(col_hbm.at[w], cidx)

    zero16 = jnp.zeros((16,), jnp.float32)

    @pl.loop(0, N // 16)
    def _(i):
        deg[pl.ds(i * 16, 16)] = zero16

    ones16 = jnp.ones((16,), jnp.float32)

    @pl.loop(0, NCH)
    def _(j):
        @pl.loop(0, K // 16)
        def _(i):
            sl = pl.ds(i * 16, 16)
            r16 = ridx[j, sl]
            c16 = cidx[j, sl]
            plsc.addupdate_scatter(deg, [r16], ones16, mask=r16 != c16)

    @pl.loop(0, N // RBLK)
    def _(g):
        pltpu.sync_copy(deg.at[pl.ds(g * RBLK, RBLK)], dall_hbm.at[g, w])


@functools.partial(
    pl.kernel,
    out_type=jax.ShapeDtypeStruct((NC, N, F), jnp.float32),
    mesh=_mesh,
    compiler_params=_sc_params,
    scratch_types=[
        pltpu.VMEM_SHARED((NJUNK, F), jnp.float32),
        pltpu.VMEM((NCH, K), jnp.int32),
        pltpu.VMEM((NCH, K), jnp.int32),
        pltpu.VMEM((K, F), jnp.float32),
        pltpu.VMEM((K, F), jnp.float32),
        pltpu.VMEM((K, F), jnp.float32),
        pltpu.SemaphoreType.DMA,
        pltpu.SemaphoreType.DMA,
        pltpu.SemaphoreType.DMA,
        pltpu.SemaphoreType.DMA,
        pltpu.SemaphoreType.DMA,
        pltpu.SemaphoreType.DMA,
    ],
)
def _conv_kernel(row_hbm, col_hbm, y_hbm, p_hbm,
                 acc, ridx, cidx, buf0, buf1, buf2,
                 gsem0, gsem1, gsem2, ssem0, ssem1, ssem2):
    c = lax.axis_index("c")
    s = lax.axis_index("s")
    w = s * NC + c
    pltpu.sync_copy(row_hbm.at[w], ridx)
    pltpu.sync_copy(col_hbm.at[w], cidx)

    njunk16 = jnp.full((16,), N, jnp.int32)

    @pl.loop(0, NCH)
    def _(j):
        @pl.loop(0, K // 16)
        def _(i):
            sl = pl.ds(i * 16, 16)
            r16 = ridx[j, sl]
            c16 = cidx[j, sl]
            cidx[j, sl] = jnp.where(r16 == c16, njunk16, c16)

    zero16 = jnp.zeros((16,), jnp.float32)

    @pl.loop(0, K)
    def _(e):
        @pl.loop(0, F // 16)
        def _(i):
            buf0[e, pl.ds(i * 16, 16)] = zero16

    base = s * RPS
    nfull = RPS // K
    rem = RPS - nfull * K

    @pl.loop(0, nfull)
    def _(i):
        pltpu.sync_copy(buf0, acc.at[pl.ds(base + i * K, K)])

    pltpu.sync_copy(
        buf0.at[pl.ds(0, rem)], acc.at[pl.ds(base + nfull * K, rem)]
    )

    bufs = (buf0, buf1, buf2)
    gsems = (gsem0, gsem1, gsem2)
    ssems = (ssem0, ssem1, ssem2)
    pltpu.async_copy(y_hbm.at[ridx.at[0]], buf0, gsem0)
    pltpu.async_copy(y_hbm.at[ridx.at[1]], buf1, gsem1)
    plsc.subcore_barrier()

    NGRP = (NCH - 2) // 3

    @pl.loop(0, NGRP)
    def _(t):
        for b in range(3):
            j = 3 * t + b
            bn = (b + 2) % 3
            pltpu.make_async_copy(y_hbm.at[ridx.at[j]], bufs[b], gsems[b]).wait()

            @pl.when(j >= 1)
            def _():
                pltpu.make_async_copy(
                    bufs[bn], acc.at[cidx.at[j - 1]], ssems[bn]
                ).wait()

            pltpu.async_copy(y_hbm.at[ridx.at[j + 2]], bufs[bn], gsems[bn])
            pltpu.async_copy(bufs[b], acc.at[cidx.at[j]], ssems[b], add=True)

    for j in (NCH - 2, NCH - 1):
        b = j % 3
        bn = (b + 2) % 3
        pltpu.make_async_copy(y_hbm.at[ridx.at[j]], bufs[b], gsems[b]).wait()
        pltpu.make_async_copy(bufs[bn], acc.at[cidx.at[j - 1]], ssems[bn]).wait()
        pltpu.async_copy(bufs[b], acc.at[cidx.at[j]], ssems[b], add=True)

    pltpu.make_async_copy(
        bufs[(NCH - 1) % 3], acc.at[cidx.at[NCH - 1]], ssems[(NCH - 1) % 3]
    ).wait()
    plsc.subcore_barrier()

    @pl.loop(0, nfull)
    def _(i):
        sl = pl.ds(base + i * K, K)
        pltpu.sync_copy(acc.at[sl], p_hbm.at[c, sl])

    sl = pl.ds(base + nfull * K, rem)
    pltpu.sync_copy(acc.at[sl], p_hbm.at[c, sl])


def _firsts_from_counts(cnt):
    lower = (
        lax.broadcasted_iota(jnp.int32, (B, B), 1)
        < lax.broadcasted_iota(jnp.int32, (B, B), 0)
    ).astype(jnp.float32)
    return lax.dot_general(
        lower, cnt, (((1,), (0,)), ((), ())), preferred_element_type=jnp.float32
    )


def _pre_body(x_ref, w1_ref, dall_ref, batch_ref, w2b_ref,
              ys_ref, cnt_ref, dis_ref, r_ref, xf_ref):
    i = pl.program_id(0)

    @pl.when(i < G)
    def _():
        deg = jnp.sum(dall_ref[...], axis=(0, 1)) + 1.0
        dis = lax.rsqrt(deg).reshape(RBLK, 1)
        dis_ref[...] = dis
        y = jnp.dot(x_ref[...], w1_ref[...],
                    preferred_element_type=jnp.float32)
        ys_ref[...] = y * dis
        seg = lax.broadcasted_iota(jnp.int32, (RBLK, B), 1)
        m = (batch_ref[...] == seg).astype(jnp.float32)
        csum = lax.dot_general(
            m,
            jnp.ones((RBLK, 1), jnp.float32),
            (((0,), (0,)), ((), ())),
            preferred_element_type=jnp.float32,
        )

        @pl.when(i == 0)
        def _():
            cnt_ref[...] = csum

        @pl.when(i > 0)
        def _():
            cnt_ref[...] += csum

    @pl.when(i >= G)
    def _():
        firsts = _firsts_from_counts(cnt_ref[...]).astype(jnp.int32)
        rowid = (i - G) * RBLK + lax.broadcasted_iota(jnp.int32, (B, RBLK), 1)
        fsel = (rowid == firsts).astype(jnp.float32)
        part = jnp.dot(fsel, x_ref[...], preferred_element_type=jnp.float32)

        @pl.when(i == G)
        def _():
            xf_ref[...] = part

        @pl.when(i > G)
        def _():
            xf_ref[...] += part

        @pl.when(i == 2 * G - 1)
        def _():
            r_ref[...] = jnp.dot(
                jnp.maximum(xf_ref[...], 0.0),
                w2b_ref[...],
                preferred_element_type=jnp.float32,
            )


def _pre(x, W1, dall, batchcol, W2b):
    return pl.pallas_call(
        _pre_body,
        grid=(2 * G,),
        in_specs=[
            pl.BlockSpec((RBLK, F), lambda i: (i % G, 0)),
            pl.BlockSpec((F, F), lambda i: (0, 0)),
            pl.BlockSpec((1, NW, RBLK), lambda i: (i % G, 0, 0)),
            pl.BlockSpec((RBLK, 1), lambda i: (i % G, 0)),
            pl.BlockSpec((F, F), lambda i: (0, 0)),
        ],
        out_specs=[
            pl.BlockSpec((RBLK, F), lambda i: (jnp.minimum(i, G - 1), 0)),
            pl.BlockSpec((B, 1), lambda i: (0, 0)),
            pl.BlockSpec((RBLK, 1), lambda i: (jnp.minimum(i, G - 1), 0)),
            pl.BlockSpec((B, F), lambda i: (0, 0)),
        ],
        out_shape=[
            jax.ShapeDtypeStruct((N, F), jnp.float32),
            jax.ShapeDtypeStruct((B, 1), jnp.float32),
            jax.ShapeDtypeStruct((N, 1), jnp.float32),
            jax.ShapeDtypeStruct((B, F), jnp.float32),
        ],
        scratch_shapes=[pltpu.VMEM((B, F), jnp.float32)],
    )(x, W1, dall, batchcol, W2b)


def _mid_body(p_ref, ys_ref, dis_ref, b1_ref, r_ref, batch_ref, w2a_ref,
              h1_ref, ys2_ref):
    dis = dis_ref[...]
    h1 = dis * (p_ref[0] + p_ref[1] + ys_ref[...]) + b1_ref[...]
    h1_ref[...] = h1
    seg = lax.broadcasted_iota(jnp.int32, (RBLK, B), 1)
    m = (batch_ref[...] == seg).astype(jnp.float32)
    rblk = jnp.dot(m, r_ref[...], preferred_element_type=jnp.float32)
    u = (
        jnp.dot(jnp.maximum(h1, 0.0), w2a_ref[...],
                preferred_element_type=jnp.float32)
        + rblk
    )
    ys2_ref[...] = u * dis


def _mid(p1, ys1, discol, b1r, r, batchcol, W2a):
    return pl.pallas_call(
        _mid_body,
        grid=(G,),
        in_specs=[
            pl.BlockSpec((NC, RBLK, F), lambda i: (0, i, 0)),
            pl.BlockSpec((RBLK, F), lambda i: (i, 0)),
            pl.BlockSpec((RBLK, 1), lambda i: (i, 0)),
            pl.BlockSpec((1, F), lambda i: (0, 0)),
            pl.BlockSpec((B, F), lambda i: (0, 0)),
            pl.BlockSpec((RBLK, 1), lambda i: (i, 0)),
            pl.BlockSpec((F, F), lambda i: (0, 0)),
        ],
        out_specs=[
            pl.BlockSpec((RBLK, F), lambda i: (i, 0)),
            pl.BlockSpec((RBLK, F), lambda i: (i, 0)),
        ],
        out_shape=[
            jax.ShapeDtypeStruct((N, F), jnp.float32),
            jax.ShapeDtypeStruct((N, F), jnp.float32),
        ],
    )(p1, ys1, discol, b1r, r, batchcol, W2a)


def _final_body(p_ref, ys2_ref, dis_ref, b2_ref, h1_ref, batch_ref, cnt_ref,
                out_ref, s_ref, t_ref):
    i = pl.program_id(0)
    h2 = dis_ref[...] * (p_ref[0] + p_ref[1] + ys2_ref[...]) + b2_ref[...]
    g = jnp.maximum(h2, 0.0)
    seg = lax.broadcasted_iota(jnp.int32, (RBLK, B), 1)
    m = (batch_ref[...] == seg).astype(jnp.float32)
    spart = lax.dot_general(
        m, g, (((0,), (0,)), ((), ())), preferred_element_type=jnp.float32
    )
    firsts = _firsts_from_counts(cnt_ref[...]).astype(jnp.int32)
    rowid = i * RBLK + lax.broadcasted_iota(jnp.int32, (B, RBLK), 1)
    fsel = (rowid == firsts).astype(jnp.float32)
    tpart = jnp.dot(fsel, h1_ref[...], preferred_element_type=jnp.float32)

    @pl.when(i == 0)
    def _():
        s_ref[...] = spart
        t_ref[...] = tpart

    @pl.when(i > 0)
    def _():
        s_ref[...] += spart
        t_ref[...] += tpart

    @pl.when(i == G - 1)
    def _():
        cnt = cnt_ref[...]
        mean = s_ref[...] / jnp.maximum(cnt, 1.0)
        root = jnp.where(cnt > 0.0, t_ref[...], 0.0)
        out_ref[...] = jnp.concatenate([mean, root], axis=1)


def _final(p2, ys2, discol, b2r, h1, batchcol, cnt):
    return pl.pallas_call(
        _final_body,
        grid=(G,),
        in_specs=[
            pl.BlockSpec((NC, RBLK, F), lambda i: (0, i, 0)),
            pl.BlockSpec((RBLK, F), lambda i: (i, 0)),
            pl.BlockSpec((RBLK, 1), lambda i: (i, 0)),
            pl.BlockSpec((1, F), lambda i: (0, 0)),
            pl.BlockSpec((RBLK, F), lambda i: (i, 0)),
            pl.BlockSpec((RBLK, 1), lambda i: (i, 0)),
            pl.BlockSpec((B, 1), lambda i: (0, 0)),
        ],
        out_specs=pl.BlockSpec((B, 2 * F), lambda i: (0, 0)),
        out_shape=jax.ShapeDtypeStruct((B, 2 * F), jnp.float32),
        scratch_shapes=[
            pltpu.VMEM((B, F), jnp.float32),
            pltpu.VMEM((B, F), jnp.float32),
        ],
    )(p2, ys2, discol, b2r, h1, batchcol, cnt)


def kernel(x, W1, b1, W2, b2, edge_index, batch):
    row3 = edge_index[0].reshape(NW, NCH, K)
    col3 = edge_index[1].reshape(NW, NCH, K)
    batchcol = batch.reshape(N, 1)
    b1r = b1.reshape(1, F)
    b2r = b2.reshape(1, F)
    W2a = W2[:F]
    W2b = W2[F:]

    dall = _deg_kernel(row3, col3)
    ys1, cnt, discol, r = _pre(x, W1, dall, batchcol, W2b)
    p1 = _conv_kernel(row3, col3, ys1)
    h1, ys2 = _mid(p1, ys1, discol, b1r, r, batchcol, W2a)
    p2 = _conv_kernel(row3, col3, ys2)
    return _final(p2, ys2, discol, b2r, h1, batchcol, cnt)


# speedup vs baseline: 1.0102x; 1.0102x over previous
import functools

import jax
import jax.numpy as jnp
from jax import lax
from jax.experimental import pallas as pl
from jax.experimental.pallas import tpu as pltpu
from jax.experimental.pallas import tpu_sc as plsc

N = 10000
E = 320000
F = 128
B = 128
NC = 2
NS = 16
NW = NC * NS
EPW = E // NW
K = 80
NCH = EPW // K
RPS = N // NS
NJUNK = N + 16
RBLK = 2000
G = N // RBLK

_mesh = plsc.VectorSubcoreMesh(core_axis_name="c", subcore_axis_name="s")
_sc_params = pltpu.CompilerParams(
    needs_layout_passes=False, use_tc_tiling_on_sc=False
)


@functools.partial(
    pl.kernel,
    out_type=jax.ShapeDtypeStruct((N // RBLK, NW, RBLK), jnp.float32),
    mesh=_mesh,
    compiler_params=_sc_params,
    scratch_types=[
        pltpu.VMEM((NCH, K), jnp.int32),
        pltpu.VMEM((NCH, K), jnp.int32),
        pltpu.VMEM((N,), jnp.float32),
    ],
)
def _deg_kernel(row_hbm, col_hbm, dall_hbm, ridx, cidx, deg):
    c = lax.axis_index("c")
    s = lax.axis_index("s")
    w = s * NC + c
    pltpu.sync_copy(row_hbm.at[w], ridx)
    pltpu.sync_copy(col_hbm.at[w], cidx)

    zero16 = jnp.zeros((16,), jnp.float32)

    @pl.loop(0, N // 16)
    def _(i):
        deg[pl.ds(i * 16, 16)] = zero16

    ones16 = jnp.ones((16,), jnp.float32)

    @pl.loop(0, NCH)
    def _(j):
        @pl.loop(0, K // 16)
        def _(i):
            sl = pl.ds(i * 16, 16)
            r16 = ridx[j, sl]
            c16 = cidx[j, sl]
            plsc.addupdate_scatter(deg, [r16], ones16, mask=r16 != c16)

    @pl.loop(0, N // RBLK)
    def _(g):
        pltpu.sync_copy(deg.at[pl.ds(g * RBLK, RBLK)], dall_hbm.at[g, w])


@functools.partial(
    pl.kernel,
    out_type=jax.ShapeDtypeStruct((NC, N, F), jnp.float32),
    mesh=_mesh,
    compiler_params=_sc_params,
    scratch_types=[
        pltpu.VMEM_SHARED((NJUNK, F), jnp.float32),
        pltpu.VMEM((NCH, K), jnp.int32),
        pltpu.VMEM((NCH, K), jnp.int32),
        pltpu.VMEM((K, F), jnp.float32),
        pltpu.VMEM((K, F), jnp.float32),
        pltpu.VMEM((K, F), jnp.float32),
        pltpu.SemaphoreType.DMA,
        pltpu.SemaphoreType.DMA,
        pltpu.SemaphoreType.DMA,
        pltpu.SemaphoreType.DMA,
        pltpu.SemaphoreType.DMA,
        pltpu.SemaphoreType.DMA,
    ],
)
def _conv_kernel(row_hbm, col_hbm, y_hbm, p_hbm,
                 acc, ridx, cidx, buf0, buf1, buf2,
                 gsem0, gsem1, gsem2, ssem0, ssem1, ssem2):
    c = lax.axis_index("c")
    s = lax.axis_index("s")
    w = s * NC + c
    pltpu.sync_copy(row_hbm.at[w], ridx)
    pltpu.sync_copy(col_hbm.at[w], cidx)

    njunk16 = jnp.full((16,), N, jnp.int32)

    @pl.loop(0, NCH)
    def _(j):
        @pl.loop(0, K // 16)
        def _(i):
            sl = pl.ds(i * 16, 16)
            r16 = ridx[j, sl]
            c16 = cidx[j, sl]
            cidx[j, sl] = jnp.where(r16 == c16, njunk16, c16)

    zero16 = jnp.zeros((16,), jnp.float32)

    @pl.loop(0, K)
    def _(e):
        @pl.loop(0, F // 16)
        def _(i):
            buf0[e, pl.ds(i * 16, 16)] = zero16

    base = s * RPS
    nfull = RPS // K
    rem = RPS - nfull * K

    @pl.loop(0, nfull)
    def _(i):
        pltpu.sync_copy(buf0, acc.at[pl.ds(base + i * K, K)])

    pltpu.sync_copy(
        buf0.at[pl.ds(0, rem)], acc.at[pl.ds(base + nfull * K, rem)]
    )

    bufs = (buf0, buf1, buf2)
    gsems = (gsem0, gsem1, gsem2)
    ssems = (ssem0, ssem1, ssem2)
    pltpu.async_copy(y_hbm.at[ridx.at[0]], buf0, gsem0)
    pltpu.async_copy(y_hbm.at[ridx.at[1]], buf1, gsem1)
    plsc.subcore_barrier()

    NGRP = (NCH - 2) // 3

    @pl.loop(0, NGRP)
    def _(t):
        for b in range(3):
            j = 3 * t + b
            bn = (b + 2) % 3
            pltpu.make_async_copy(y_hbm.at[ridx.at[j]], bufs[b], gsems[b]).wait()

            @pl.when(j >= 1)
            def _():
                pltpu.make_async_copy(
                    bufs[bn], acc.at[cidx.at[j - 1]], ssems[bn]
                ).wait()

            pltpu.async_copy(y_hbm.at[ridx.at[j + 2]], bufs[bn], gsems[bn])
            pltpu.async_copy(bufs[b], acc.at[cidx.at[j]], ssems[b], add=True)

    for j in (NCH - 2, NCH - 1):
        b = j % 3
        bn = (b + 2) % 3
        pltpu.make_async_copy(y_hbm.at[ridx.at[j]], bufs[b], gsems[b]).wait()
        pltpu.make_async_copy(bufs[bn], acc.at[cidx.at[j - 1]], ssems[bn]).wait()
        pltpu.async_copy(bufs[b], acc.at[cidx.at[j]], ssems[b], add=True)

    pltpu.make_async_copy(
        bufs[(NCH - 1) % 3], acc.at[cidx.at[NCH - 1]], ssems[(NCH - 1) % 3]
    ).wait()
    plsc.subcore_barrier()

    @pl.loop(0, nfull)
    def _(i):
        sl = pl.ds(base + i * K, K)
        pltpu.sync_copy(acc.at[sl], p_hbm.at[c, sl])

    sl = pl.ds(base + nfull * K, rem)
    pltpu.sync_copy(acc.at[sl], p_hbm.at[c, sl])


def _firsts_from_counts(cnt):
    lower = (
        lax.broadcasted_iota(jnp.int32, (B, B), 1)
        < lax.broadcasted_iota(jnp.int32, (B, B), 0)
    ).astype(jnp.float32)
    return lax.dot_general(
        lower, cnt, (((1,), (0,)), ((), ())), preferred_element_type=jnp.float32
    )


def _y1cnt_body(x_ref, w1_ref, batch_ref, y_ref, cnt_ref):
    i = pl.program_id(0)
    y_ref[...] = jnp.dot(x_ref[...], w1_ref[...],
                         preferred_element_type=jnp.float32)
    seg = lax.broadcasted_iota(jnp.int32, (RBLK, B), 1)
    m = (batch_ref[...] == seg).astype(jnp.float32)
    csum = lax.dot_general(
        m,
        jnp.ones((RBLK, 1), jnp.float32),
        (((0,), (0,)), ((), ())),
        preferred_element_type=jnp.float32,
    )

    @pl.when(i == 0)
    def _():
        cnt_ref[...] = csum

    @pl.when(i > 0)
    def _():
        cnt_ref[...] += csum


def _y1cnt(x, W1, batchcol):
    return pl.pallas_call(
        _y1cnt_body,
        grid=(G,),
        in_specs=[
            pl.BlockSpec((RBLK, F), lambda i: (i, 0)),
            pl.BlockSpec((F, F), lambda i: (0, 0)),
            pl.BlockSpec((RBLK, 1), lambda i: (i, 0)),
        ],
        out_specs=[
            pl.BlockSpec((RBLK, F), lambda i: (i, 0)),
            pl.BlockSpec((B, 1), lambda i: (0, 0)),
        ],
        out_shape=[
            jax.ShapeDtypeStruct((N, F), jnp.float32),
            jax.ShapeDtypeStruct((B, 1), jnp.float32),
        ],
    )(x, W1, batchcol)


def _scale_root_body(y_ref, dall_ref, x_ref, cnt_ref, w2b_ref,
                     ys_ref, dis_ref, r_ref, xf_ref):
    i = pl.program_id(0)
    deg = jnp.sum(dall_ref[...], axis=(0, 1)) + 1.0
    dis = lax.rsqrt(deg).reshape(RBLK, 1)
    dis_ref[...] = dis
    ys_ref[...] = y_ref[...] * dis
    firsts = _firsts_from_counts(cnt_ref[...]).astype(jnp.int32)
    rowid = i * RBLK + lax.broadcasted_iota(jnp.int32, (B, RBLK), 1)
    fsel = (rowid == firsts).astype(jnp.float32)
    part = jnp.dot(fsel, x_ref[...], preferred_element_type=jnp.float32)

    @pl.when(i == 0)
    def _():
        xf_ref[...] = part

    @pl.when(i > 0)
    def _():
        xf_ref[...] += part

    @pl.when(i == G - 1)
    def _():
        r_ref[...] = jnp.dot(
            jnp.maximum(xf_ref[...], 0.0),
            w2b_ref[...],
            preferred_element_type=jnp.float32,
        )


def _scale_root(y1, dall, x, cnt, W2b):
    return pl.pallas_call(
        _scale_root_body,
        grid=(G,),
        in_specs=[
            pl.BlockSpec((RBLK, F), lambda i: (i, 0)),
            pl.BlockSpec((1, NW, RBLK), lambda i: (i, 0, 0)),
            pl.BlockSpec((RBLK, F), lambda i: (i, 0)),
            pl.BlockSpec((B, 1), lambda i: (0, 0)),
            pl.BlockSpec((F, F), lambda i: (0, 0)),
        ],
        out_specs=[
            pl.BlockSpec((RBLK, F), lambda i: (i, 0)),
            pl.BlockSpec((RBLK, 1), lambda i: (i, 0)),
            pl.BlockSpec((B, F), lambda i: (0, 0)),
        ],
        out_shape=[
            jax.ShapeDtypeStruct((N, F), jnp.float32),
            jax.ShapeDtypeStruct((N, 1), jnp.float32),
            jax.ShapeDtypeStruct((B, F), jnp.float32),
        ],
        scratch_shapes=[pltpu.VMEM((B, F), jnp.float32)],
    )(y1, dall, x, cnt, W2b)


def _mid_body(p_ref, ys_ref, dis_ref, b1_ref, r_ref, batch_ref, w2a_ref,
              h1_ref, ys2_ref):
    dis = dis_ref[...]
    h1 = dis * (p_ref[0] + p_ref[1] + ys_ref[...]) + b1_ref[...]
    h1_ref[...] = h1
    seg = lax.broadcasted_iota(jnp.int32, (RBLK, B), 1)
    m = (batch_ref[...] == seg).astype(jnp.float32)
    rblk = jnp.dot(m, r_ref[...], preferred_element_type=jnp.float32)
    u = (
        jnp.dot(jnp.maximum(h1, 0.0), w2a_ref[...],
                preferred_element_type=jnp.float32)
        + rblk
    )
    ys2_ref[...] = u * dis


def _mid(p1, ys1, discol, b1r, r, batchcol, W2a):
    return pl.pallas_call(
        _mid_body,
        grid=(G,),
        in_specs=[
            pl.BlockSpec((NC, RBLK, F), lambda i: (0, i, 0)),
            pl.BlockSpec((RBLK, F), lambda i: (i, 0)),
            pl.BlockSpec((RBLK, 1), lambda i: (i, 0)),
            pl.BlockSpec((1, F), lambda i: (0, 0)),
            pl.BlockSpec((B, F), lambda i: (0, 0)),
            pl.BlockSpec((RBLK, 1), lambda i: (i, 0)),
            pl.BlockSpec((F, F), lambda i: (0, 0)),
        ],
        out_specs=[
            pl.BlockSpec((RBLK, F), lambda i: (i, 0)),
            pl.BlockSpec((RBLK, F), lambda i: (i, 0)),
        ],
        out_shape=[
            jax.ShapeDtypeStruct((N, F), jnp.float32),
            jax.ShapeDtypeStruct((N, F), jnp.float32),
        ],
    )(p1, ys1, discol, b1r, r, batchcol, W2a)


def _final_body(p_ref, ys2_ref, dis_ref, b2_ref, h1_ref, batch_ref, cnt_ref,
                out_ref, s_ref, t_ref):
    i = pl.program_id(0)
    h2 = dis_ref[...] * (p_ref[0] + p_ref[1] + ys2_ref[...]) + b2_ref[...]
    g = jnp.maximum(h2, 0.0)
    seg = lax.broadcasted_iota(jnp.int32, (RBLK, B), 1)
    m = (batch_ref[...] == seg).astype(jnp.float32)
    spart = lax.dot_general(
        m, g, (((0,), (0,)), ((), ())), preferred_element_type=jnp.float32
    )
    firsts = _firsts_from_counts(cnt_ref[...]).astype(jnp.int32)
    rowid = i * RBLK + lax.broadcasted_iota(jnp.int32, (B, RBLK), 1)
    fsel = (rowid == firsts).astype(jnp.float32)
    tpart = jnp.dot(fsel, h1_ref[...], preferred_element_type=jnp.float32)

    @pl.when(i == 0)
    def _():
        s_ref[...] = spart
        t_ref[...] = tpart

    @pl.when(i > 0)
    def _():
        s_ref[...] += spart
        t_ref[...] += tpart

    @pl.when(i == G - 1)
    def _():
        cnt = cnt_ref[...]
        mean = s_ref[...] / jnp.maximum(cnt, 1.0)
        root = jnp.where(cnt > 0.0, t_ref[...], 0.0)
        out_ref[...] = jnp.concatenate([mean, root], axis=1)


def _final(p2, ys2, discol, b2r, h1, batchcol, cnt):
    return pl.pallas_call(
        _final_body,
        grid=(G,),
        in_specs=[
            pl.BlockSpec((NC, RBLK, F), lambda i: (0, i, 0)),
            pl.BlockSpec((RBLK, F), lambda i: (i, 0)),
            pl.BlockSpec((RBLK, 1), lambda i: (i, 0)),
            pl.BlockSpec((1, F), lambda i: (0, 0)),
            pl.BlockSpec((RBLK, F), lambda i: (i, 0)),
            pl.BlockSpec((RBLK, 1), lambda i: (i, 0)),
            pl.BlockSpec((B, 1), lambda i: (0, 0)),
        ],
        out_specs=pl.BlockSpec((B, 2 * F), lambda i: (0, 0)),
        out_shape=jax.ShapeDtypeStruct((B, 2 * F), jnp.float32),
        scratch_shapes=[
            pltpu.VMEM((B, F), jnp.float32),
            pltpu.VMEM((B, F), jnp.float32),
        ],
    )(p2, ys2, discol, b2r, h1, batchcol, cnt)


def kernel(x, W1, b1, W2, b2, edge_index, batch):
    row3 = edge_index[0].reshape(NW, NCH, K)
    col3 = edge_index[1].reshape(NW, NCH, K)
    batchcol = batch.reshape(N, 1)
    b1r = b1.reshape(1, F)
    b2r = b2.reshape(1, F)
    W2a = W2[:F]
    W2b = W2[F:]

    dall = _deg_kernel(row3, col3)
    y1, cnt = _y1cnt(x, W1, batchcol)
    ys1, discol, r = _scale_root(y1, dall, x, cnt, W2b)
    p1 = _conv_kernel(row3, col3, ys1)
    h1, ys2 = _mid(p1, ys1, discol, b1r, r, batchcol, W2a)
    p2 = _conv_kernel(row3, col3, ys2)
    return _final(p2, ys2, discol, b2r, h1, batchcol, cnt)
